# baseline (speedup 1.0000x reference)
import functools

import jax
import jax.numpy as jnp
from jax.experimental import pallas as pl
from jax.experimental.pallas import tpu as pltpu

B, T, V = 32, 16, 100000
K = B * T
L = 128


def _body(probs_hbm, dummy_hbm, a_smem, avm_ref, out_ref, x_ref, sem):
    for k in range(K):
        b, t = k % B, k // B
        bt = b * T + t
        start = pl.multiple_of((a_smem[b, t] >> 7) << 7, L)
        pltpu.make_async_copy(
            probs_hbm.at[pl.ds(bt & ~7, 8), pl.ds(start, L)],
            x_ref.at[:, k, :],
            sem,
        ).start()
    pltpu.make_async_copy(dummy_hbm, x_ref, sem).wait()

    lanes = jax.lax.broadcasted_iota(jnp.int32, (B, L), 1)
    p_cols = []
    m_cols = []
    for t in range(T):
        at = avm_ref[:, t:t + 1]
        xt = x_ref[t & 7, t * B:(t + 1) * B, :]
        p = jnp.sum(jnp.where(lanes == (at & 127), xt, 0.0),
                    axis=1, keepdims=True)
        p_cols.append(p)
        m_cols.append(jnp.where((at != 0) & (at != 1), 1.0, 0.0))
    pmat = jnp.concatenate(p_cols, axis=1)
    mmat = jnp.concatenate(m_cols, axis=1).astype(jnp.float32)
    term = -jnp.log(pmat) * mmat
    loss = (jnp.sum(term, axis=1, keepdims=True)
            / jnp.sum(mmat, axis=1, keepdims=True))
    out_ref[...] = jnp.squeeze(loss, axis=1)


@functools.partial(jax.jit, static_argnames=())
def _decoder_loss_tc(probs2, dummy, a_trg):
    return pl.pallas_call(
        _body,
        out_shape=jax.ShapeDtypeStruct((B,), jnp.float32),
        in_specs=[
            pl.BlockSpec(memory_space=pl.ANY),
            pl.BlockSpec(memory_space=pl.ANY),
            pl.BlockSpec(memory_space=pltpu.SMEM),
            pl.BlockSpec(memory_space=pltpu.VMEM),
        ],
        out_specs=pl.BlockSpec(memory_space=pltpu.VMEM),
        scratch_shapes=[
            pltpu.VMEM((8, K, L), jnp.float32),
            pltpu.SemaphoreType.DMA,
        ],
    )(probs2, dummy, a_trg, a_trg)


def kernel(probs, a_trg):
    probs2 = probs.reshape(B * T, V)
    dummy = jnp.zeros((8, K, L), jnp.float32)
    return _decoder_loss_tc(probs2, dummy, a_trg)


# speedup vs baseline: 1.2048x; 1.2048x over previous
import functools

import jax
import jax.numpy as jnp
from jax.experimental import pallas as pl
from jax.experimental.pallas import tpu as pltpu

B, T, V = 32, 16, 100000
K = B * T
L = 128


def _body(probs_hbm, a_smem, avm_ref, out_ref, x_ref, sem):
    copies = []
    for k in range(K):
        b, t = k % B, k // B
        bt = b * T + t
        start = pl.multiple_of((a_smem[b, t] >> 7) << 7, L)
        cp = pltpu.make_async_copy(
            probs_hbm.at[pl.ds(bt & ~7, 8), pl.ds(start, L)],
            x_ref.at[:, k, :],
            sem,
        )
        cp.start()
        copies.append(cp)
    for cp in copies:
        cp.wait()

    lanes = jax.lax.broadcasted_iota(jnp.int32, (B, L), 1)
    p_cols = []
    m_cols = []
    for t in range(T):
        at = avm_ref[:, t:t + 1]
        xt = x_ref[t & 7, t * B:(t + 1) * B, :]
        p = jnp.sum(jnp.where(lanes == (at & 127), xt, 0.0),
                    axis=1, keepdims=True)
        p_cols.append(p)
        m_cols.append(jnp.where((at != 0) & (at != 1), 1.0, 0.0))
    pmat = jnp.concatenate(p_cols, axis=1)
    mmat = jnp.concatenate(m_cols, axis=1).astype(jnp.float32)
    term = -jnp.log(pmat) * mmat
    loss = (jnp.sum(term, axis=1, keepdims=True)
            / jnp.sum(mmat, axis=1, keepdims=True))
    out_ref[...] = jnp.squeeze(loss, axis=1)


@functools.partial(jax.jit, static_argnames=())
def _decoder_loss_tc(probs2, a_trg):
    return pl.pallas_call(
        _body,
        out_shape=jax.ShapeDtypeStruct((B,), jnp.float32),
        in_specs=[
            pl.BlockSpec(memory_space=pl.ANY),
            pl.BlockSpec(memory_space=pltpu.SMEM),
            pl.BlockSpec(memory_space=pltpu.VMEM),
        ],
        out_specs=pl.BlockSpec(memory_space=pltpu.VMEM),
        scratch_shapes=[
            pltpu.VMEM((8, K, L), jnp.float32),
            pltpu.SemaphoreType.DMA,
        ],
    )(probs2, a_trg, a_trg)


def kernel(probs, a_trg):
    probs2 = probs.reshape(B * T, V)
    return _decoder_loss_tc(probs2, a_trg)
